# baseline (speedup 1.0000x reference)
import jax
import jax.numpy as jnp
from jax import lax
from jax.experimental import pallas as pl
from jax.experimental.pallas import tpu as pltpu
from jax.experimental.pallas import tpu_sc as plsc

CARDINALITY = 1000000
EMBED_DIM = 64
BATCH = 16384

NUM_CORES = 2
NUM_SUBCORES = 16
NUM_WORKERS = NUM_CORES * NUM_SUBCORES
B_PER_W = BATCH // NUM_WORKERS
CHUNK = 128
NCHUNK = B_PER_W // CHUNK
LANES = 16


def _gather_body(t128_hbm, idx_hbm, out_hbm, idx_v, q_v, rows_v, sem, osem):
    wid = lax.axis_index("s") * NUM_CORES + lax.axis_index("c")
    base = wid * B_PER_W

    pltpu.sync_copy(idx_hbm.at[pl.ds(base, B_PER_W)], idx_v)

    def make_q(i, carry):
        v = idx_v[pl.ds(i * LANES, LANES)]
        jr = i // (CHUNK // LANES)
        kr = (i % (CHUNK // LANES)) * LANES
        q_v[jr, pl.ds(kr, LANES)] = v
        return carry

    lax.fori_loop(0, B_PER_W // LANES, make_q, 0)

    copies = [
        pltpu.async_copy(
            t128_hbm.at[q_v.at[j]],
            rows_v.at[pl.ds(j * CHUNK, CHUNK)],
            sem,
        )
        for j in range(NCHUNK)
    ]
    for c in copies:
        c.wait()

    pltpu.async_copy(rows_v, out_hbm.at[pl.ds(base, B_PER_W)], osem).wait()


@jax.jit
def _sc_gather(table, idx):
    mesh = plsc.VectorSubcoreMesh(core_axis_name="c", subcore_axis_name="s")
    fn = pl.kernel(
        _gather_body,
        mesh=mesh,
        out_type=jax.ShapeDtypeStruct((BATCH, 2 * EMBED_DIM), jnp.float32),
        scratch_types=[
            pltpu.VMEM((B_PER_W,), jnp.int32),
            pltpu.VMEM((NCHUNK, CHUNK), jnp.int32),
            pltpu.VMEM((B_PER_W, 2 * EMBED_DIM), jnp.float32),
            pltpu.SemaphoreType.DMA,
            pltpu.SemaphoreType.DMA,
        ],
    )
    t_pad = jnp.concatenate(
        [table, jnp.zeros((CARDINALITY, EMBED_DIM), jnp.float32)], axis=1
    )
    out128 = fn(t_pad, idx)
    return out128[:, :EMBED_DIM]


def kernel(x, table):
    return _sc_gather(table, x.astype(jnp.int32))


# speedup vs baseline: 1.8696x; 1.8696x over previous
import jax
import jax.numpy as jnp
from jax import lax
from jax.experimental import pallas as pl
from jax.experimental.pallas import tpu as pltpu
from jax.experimental.pallas import tpu_sc as plsc

CARDINALITY = 1000000
EMBED_DIM = 64
BATCH = 16384

NUM_CORES = 2
NUM_SUBCORES = 16
NUM_WORKERS = NUM_CORES * NUM_SUBCORES
B_PER_W = BATCH // NUM_WORKERS
GROUP = 8
NGROUP = B_PER_W // GROUP
BLOCK = 64
LANES = 16
STAGE_W = EMBED_DIM * 128


def _gather_body(t3_hbm, idx_hbm, out_hbm, idx_v, stage_v, rout_v, sem, osem):
    wid = lax.axis_index("s") * NUM_CORES + lax.axis_index("c")
    base = wid * B_PER_W

    pltpu.sync_copy(idx_hbm.at[pl.ds(base, B_PER_W)], idx_v)

    iota = lax.iota(jnp.int32, LANES)

    def do_group(g2, carry):
        rv = idx_v[pl.ds(g2 * LANES, LANES)]
        kv = rv & 127
        tbv = (rv >> 7) << 7

        for half in range(2):
            for l in range(GROUP):
                tb = pl.multiple_of(tbv[half * GROUP + l], 128)
                for c_hi in range(8):
                    for c_lo in range(8):
                        c = c_hi * 8 + c_lo
                        pltpu.async_copy(
                            t3_hbm.at[c_hi, c_lo, pl.ds(tb, 128)],
                            stage_v.at[pl.ds(l * STAGE_W + c * 128, 128)],
                            sem,
                        )

            pltpu.make_async_copy(
                t3_hbm.at[0, 0, pl.ds(0, GROUP * STAGE_W)],
                stage_v,
                sem,
            ).wait()

            g = g2 * 2 + half
            blk = g // (BLOCK // GROUP)
            buf = blk % 2
            for l in range(GROUP):
                lo = (g % (BLOCK // GROUP)) * GROUP + l
                kk = iota * 0 + kv[half * GROUP + l]
                for cg in range(EMBED_DIM // LANES):
                    src_idx = l * STAGE_W + (cg * LANES + iota) * 128 + kk
                    vals = plsc.load_gather(stage_v, [src_idx])
                    rout_v[buf, lo, pl.ds(cg * LANES, LANES)] = vals

            if half == 1:
                @pl.when(g % (BLOCK // GROUP) == (BLOCK // GROUP) - 1)
                def _(blk=blk, buf=buf):
                    pltpu.async_copy(
                        rout_v.at[buf],
                        out_hbm.at[pl.ds(base + blk * BLOCK, BLOCK)],
                        osem,
                    ).wait()

        return carry

    lax.fori_loop(0, NGROUP // 2, do_group, 0)


@jax.jit
def _sc_gather(table, idx):
    mesh = plsc.VectorSubcoreMesh(core_axis_name="c", subcore_axis_name="s")
    fn = pl.kernel(
        _gather_body,
        mesh=mesh,
        out_type=jax.ShapeDtypeStruct((BATCH, 2 * EMBED_DIM), jnp.float32),
        scratch_types=[
            pltpu.VMEM((B_PER_W,), jnp.int32),
            pltpu.VMEM((GROUP * STAGE_W,), jnp.float32),
            pltpu.VMEM((2, BLOCK, 2 * EMBED_DIM), jnp.float32),
            pltpu.SemaphoreType.DMA,
            pltpu.SemaphoreType.DMA,
        ],
        compiler_params=pltpu.CompilerParams(needs_layout_passes=False),
    )
    t3 = table.T.reshape(8, 8, CARDINALITY)
    out128 = fn(t3, idx)
    return out128[:, :EMBED_DIM]


def kernel(x, table):
    return _sc_gather(table, x.astype(jnp.int32))


# speedup vs baseline: 2.0994x; 1.1229x over previous
import jax
import jax.numpy as jnp
from jax import lax
from jax.experimental import pallas as pl
from jax.experimental.pallas import tpu as pltpu
from jax.experimental.pallas import tpu_sc as plsc

CARDINALITY = 1000000
EMBED_DIM = 64
BATCH = 16384

NUM_CORES = 2
NUM_SUBCORES = 16
NUM_WORKERS = NUM_CORES * NUM_SUBCORES
B_PER_W = BATCH // NUM_WORKERS
GROUP = 8
NGROUP = B_PER_W // GROUP
BLOCK = 64
LANES = 16
STAGE_W = EMBED_DIM * 128


def _gather_body(t3_hbm, idx_hbm, out_hbm, idx_v, stage_v, rout_v, sem, osem):
    wid = lax.axis_index("s") * NUM_CORES + lax.axis_index("c")
    base = wid * B_PER_W

    pltpu.sync_copy(idx_hbm.at[pl.ds(base, B_PER_W)], idx_v)

    iota = lax.iota(jnp.int32, LANES)

    def do_group(g2, carry):
        rv = idx_v[pl.ds(g2 * LANES, LANES)]
        kv = rv & 127
        tbv = (rv >> 7) << 7

        for half in range(2):
            for l in range(GROUP):
                tb = pl.multiple_of(tbv[half * GROUP + l], 128)
                for c_hi in range(8):
                    pltpu.async_copy(
                        t3_hbm.at[c_hi, :, pl.ds(tb, 128)],
                        stage_v.at[l * 8 + c_hi],
                        sem,
                    )

            pltpu.make_async_copy(
                t3_hbm.at[0, :, pl.ds(0, 128)],
                stage_v.at[0],
                sem,
            ).wait()
            for _ in range(GROUP * 8 - 1):
                pltpu.make_async_copy(
                    t3_hbm.at[0, :, pl.ds(0, 128)],
                    stage_v.at[0],
                    sem,
                ).wait()

            g = g2 * 2 + half
            blk = g // (BLOCK // GROUP)
            buf = blk % 2
            for l in range(GROUP):
                lo = (g % (BLOCK // GROUP)) * GROUP + l
                kk = iota * 0 + kv[half * GROUP + l]
                for cg in range(EMBED_DIM // LANES):
                    cvec = cg * LANES + iota
                    vals = plsc.load_gather(
                        stage_v, [l * 8 + (cvec >> 3), cvec & 7, kk]
                    )
                    rout_v[buf, lo, pl.ds(cg * LANES, LANES)] = vals

            if half == 1:
                @pl.when(g % (BLOCK // GROUP) == (BLOCK // GROUP) - 1)
                def _(blk=blk, buf=buf):
                    pltpu.async_copy(
                        rout_v.at[buf],
                        out_hbm.at[pl.ds(base + blk * BLOCK, BLOCK)],
                        osem,
                    ).wait()

        return carry

    lax.fori_loop(0, NGROUP // 2, do_group, 0)


@jax.jit
def _sc_gather(table, idx):
    mesh = plsc.VectorSubcoreMesh(core_axis_name="c", subcore_axis_name="s")
    fn = pl.kernel(
        _gather_body,
        mesh=mesh,
        out_type=jax.ShapeDtypeStruct((BATCH, 2 * EMBED_DIM), jnp.float32),
        scratch_types=[
            pltpu.VMEM((B_PER_W,), jnp.int32),
            pltpu.VMEM((GROUP * 8, 8, 128), jnp.float32),
            pltpu.VMEM((2, BLOCK, 2 * EMBED_DIM), jnp.float32),
            pltpu.SemaphoreType.DMA,
            pltpu.SemaphoreType.DMA,
        ],
        compiler_params=pltpu.CompilerParams(needs_layout_passes=False),
    )
    t3 = table.T.reshape(8, 8, CARDINALITY)
    out128 = fn(t3, idx)
    return out128[:, :EMBED_DIM]


def kernel(x, table):
    return _sc_gather(table, x.astype(jnp.int32))
